# baseline (speedup 1.0000x reference)
import jax
import jax.numpy as jnp
from jax import lax
from jax.experimental import pallas as pl
from jax.experimental.pallas import tpu as pltpu
from jax.experimental.pallas import tpu_sc as plsc

_B, _C, _H, _W, _K, _NC = 8, 2, 128, 128, 128, 80
_HW = _H * _W
_NSLOT = _B * _K
_NCORE = 2
_NSUB = 16
_NW = _NCORE * _NSUB
_PER = _NSLOT // _NW
_NCHUNK = _PER // 16

_IV_IND = 0
_IV_INDE = _PER
_IV_IDXP = 4 * _PER
_IV_IDXH = 6 * _PER
_IV_LEN = 7 * _PER
_FV_MASK = 0
_FV_TGT = _PER
_FV_PRED = 3 * _PER
_FV_P = 5 * _PER
_FV_ACC = 6 * _PER
_FV_LEN = 7 * _PER


def _smooth_l1_vec(d):
    a = jnp.abs(d)
    return jnp.where(a < 1.0, 0.5 * a * a, a - 0.5)


def _sc_loss_kernel(out_flat, hm_flat, ind, inde_flat, mask, tgt_flat,
                    o_parts, iv, fv, accv, sem_i, sem_e, sem_j):
    cid = lax.axis_index("c")
    sid = lax.axis_index("s")
    wid = cid * _NSUB + sid
    base = wid * _PER
    b = lax.div(base, _K)

    cmi = pltpu.async_copy(ind.at[pl.ds(base, _PER)],
                           iv.at[pl.ds(_IV_IND, _PER)], sem_i)
    cme = pltpu.async_copy(inde_flat.at[pl.ds(3 * base, 3 * _PER)],
                           iv.at[pl.ds(_IV_INDE, 3 * _PER)], sem_e)
    cmm = pltpu.async_copy(mask.at[pl.ds(base, _PER)],
                           fv.at[pl.ds(_FV_MASK, _PER)], sem_j)
    cmt = pltpu.async_copy(tgt_flat.at[pl.ds(2 * base, 2 * _PER)],
                           fv.at[pl.ds(_FV_TGT, 2 * _PER)], sem_j)

    iota = lax.iota(jnp.int32, 16)
    cmi.wait()
    for c in range(_NCHUNK):
        p0 = b * (_C * _HW) + iv[pl.ds(_IV_IND + 16 * c, 16)]
        j2 = _IV_IDXP + (iota + 16 * c) * 2
        plsc.store_scatter(iv, [j2], p0)
        plsc.store_scatter(iv, [j2 + 1], p0 + _HW)
    cpp = pltpu.async_copy(out_flat.at[iv.at[pl.ds(_IV_IDXP, 2 * _PER)]],
                           fv.at[pl.ds(_FV_PRED, 2 * _PER)], sem_j)

    cme.wait()
    for c in range(_NCHUNK):
        j3 = _IV_INDE + (iota + 16 * c) * 3
        i0 = plsc.load_gather(iv, [j3])
        i1 = plsc.load_gather(iv, [j3 + 1])
        i2 = plsc.load_gather(iv, [j3 + 2])
        iv[pl.ds(_IV_IDXH + 16 * c, 16)] = (b * (_NC * _HW) + i0 * _HW
                                            + i1 * _W + i2)
    cph = pltpu.async_copy(hm_flat.at[iv.at[pl.ds(_IV_IDXH, _PER)]],
                           fv.at[pl.ds(_FV_P, _PER)], sem_j)

    cmm.wait()
    cmt.wait()
    cpp.wait()
    cph.wait()

    acc = jnp.zeros((16,), jnp.float32)
    mac = jnp.zeros((16,), jnp.float32)
    for c in range(2 * _NCHUNK):
        j = iota + 16 * c
        slot = lax.shift_right_logical(j, 1)
        s = _smooth_l1_vec(fv[pl.ds(_FV_PRED + 16 * c, 16)]
                           - fv[pl.ds(_FV_TGT + 16 * c, 16)])
        w = 1.0 + plsc.load_gather(fv, [_FV_P + slot])
        m = plsc.load_gather(fv, [_FV_MASK + slot])
        acc = acc + s * (w * w * m * 0.5)
    for c in range(_NCHUNK):
        mac = mac + fv[pl.ds(_FV_MASK + 16 * c, 16)]
    accv[0, :] = acc
    accv[1, :] = mac
    pltpu.sync_copy(accv, o_parts.at[wid])


def _tc_finish_kernel(parts_ref, out_ref):
    a = parts_ref[...]
    num = jnp.sum(a[:, 0, :])
    den = jnp.sum(a[:, 1, :])
    out_ref[...] = jnp.broadcast_to(num / den, (1, 1))


@jax.jit
def kernel(output, mask, ind, target, inde, hm):
    out_flat = output.reshape(-1)
    hm_flat = hm.reshape(-1)
    ind_f = ind.reshape(-1).astype(jnp.int32)
    inde_flat = inde.reshape(-1).astype(jnp.int32)
    mask_f = mask.reshape(-1)
    tgt_flat = target.reshape(-1)

    f32 = jnp.float32
    i32 = jnp.int32
    sc_run = pl.kernel(
        _sc_loss_kernel,
        out_type=jax.ShapeDtypeStruct((_NW, 2, 16), f32),
        mesh=plsc.VectorSubcoreMesh(core_axis_name="c", subcore_axis_name="s"),
        compiler_params=pltpu.CompilerParams(needs_layout_passes=False),
        scratch_types=[
            pltpu.VMEM((_IV_LEN,), i32),
            pltpu.VMEM((_FV_LEN,), f32),
            pltpu.VMEM((2, 16), f32),
            pltpu.SemaphoreType.DMA,
            pltpu.SemaphoreType.DMA,
            pltpu.SemaphoreType.DMA,
        ],
    )
    parts = sc_run(out_flat, hm_flat, ind_f, inde_flat, mask_f, tgt_flat)

    loss = pl.pallas_call(
        _tc_finish_kernel,
        out_shape=jax.ShapeDtypeStruct((1, 1), f32),
    )(parts)
    return loss[0, 0]
